# baseline (speedup 1.0000x reference)
import jax
import jax.numpy as jnp
from jax.experimental import pallas as pl
from jax.experimental.pallas import tpu as pltpu

_NNODES = 208
_NCLASS = 64
_NODES_PER_STEP = 16
_CHUNK = _NODES_PER_STEP * _NCLASS
_NSTEPS = _NNODES // _NODES_PER_STEP


def _fused(x_ref, adj_ref, w1_ref, b1_ref, w2_ref, b2_ref,
           fc1w_ref, fc1b_ref, fc2w_ref, fc2b_ref, fc3w_ref, fc3b_ref,
           out_ref, h2_ref, acc_ref):
    f32 = jnp.float32
    g = pl.program_id(0)

    @pl.when(g == 0)
    def _gcn():
        adj = adj_ref[...]
        s1 = jnp.dot(x_ref[...], w1_ref[...], preferred_element_type=f32)
        h1 = jnp.maximum(
            jnp.dot(adj, s1, preferred_element_type=f32) + b1_ref[...], 0.0)
        s2 = jnp.dot(h1, w2_ref[...], preferred_element_type=f32)
        h2_ref[...] = jnp.maximum(
            jnp.dot(adj, s2, preferred_element_type=f32) + b2_ref[...], 0.0)
        acc_ref[...] = jnp.zeros_like(acc_ref)

    h2g = h2_ref[pl.ds(g * _NODES_PER_STEP, _NODES_PER_STEP), :]
    col = jax.lax.broadcasted_iota(jnp.int32, (_NCLASS, _CHUNK), 1)
    tile = (col % _NCLASS == jax.lax.broadcasted_iota(
        jnp.int32, (_NCLASS, _CHUNK), 0)).astype(f32)
    expand = jnp.dot(h2g, tile, preferred_element_type=f32)
    band = (jax.lax.broadcasted_iota(jnp.int32, (_NODES_PER_STEP, _CHUNK), 1)
            // _NCLASS == jax.lax.broadcasted_iota(
                jnp.int32, (_NODES_PER_STEP, _CHUNK), 0))
    hflat = jnp.sum(jnp.where(band, expand, 0.0), axis=0, keepdims=True)

    acc_ref[...] += jax.lax.dot_general(
        hflat, fc1w_ref[...], (((1,), (1,)), ((), ())),
        preferred_element_type=f32)

    @pl.when(g == _NSTEPS - 1)
    def _head():
        f1 = jnp.maximum(acc_ref[...] + fc1b_ref[...], 0.0)
        f2 = jnp.sum(fc2w_ref[...] * f1, axis=1, keepdims=True)
        f2 = jnp.maximum(f2 + fc2b_ref[...], 0.0)
        f3 = jnp.sum(f2 * fc3w_ref[...], keepdims=True) + fc3b_ref[...]
        out_ref[...] = jax.nn.sigmoid(f3)


def kernel(x, adj, W1, b1, W2, b2, fc1_w, fc1_b, fc2_w, fc2_b, fc3_w, fc3_b):
    full = lambda a: pl.BlockSpec(a.shape, lambda g: (0,) * a.ndim)
    out = pl.pallas_call(
        _fused,
        grid=(_NSTEPS,),
        in_specs=[
            full(x), full(adj), full(W1), pl.BlockSpec((1, 256), lambda g: (0, 0)),
            full(W2), pl.BlockSpec((1, _NCLASS), lambda g: (0, 0)),
            pl.BlockSpec((128, _CHUNK), lambda g: (0, g)),
            pl.BlockSpec((1, 128), lambda g: (0, 0)),
            full(fc2_w), pl.BlockSpec((32, 1), lambda g: (0, 0)),
            pl.BlockSpec((32, 1), lambda g: (0, 0)),
            pl.BlockSpec((1, 1), lambda g: (0, 0)),
        ],
        out_specs=pl.BlockSpec((1, 1), lambda g: (0, 0)),
        out_shape=jax.ShapeDtypeStruct((1, 1), jnp.float32),
        scratch_shapes=[
            pltpu.VMEM((_NNODES, _NCLASS), jnp.float32),
            pltpu.VMEM((1, 128), jnp.float32),
        ],
        compiler_params=pltpu.CompilerParams(
            dimension_semantics=("arbitrary",)),
    )(x, adj, W1, b1.reshape(1, -1), W2, b2.reshape(1, -1),
      fc1_w, fc1_b.reshape(1, -1), fc2_w, fc2_b.reshape(-1, 1),
      fc3_w.reshape(-1, 1), fc3_b.reshape(1, 1))
    return out.reshape(1)


# speedup vs baseline: 1.2273x; 1.2273x over previous
import jax
import jax.numpy as jnp
from jax.experimental import pallas as pl
from jax.experimental.pallas import tpu as pltpu

_NNODES = 208
_NCLASS = 64
_FLAT = _NNODES * _NCLASS
_NSTEPS = 8
_CHUNK = _FLAT // _NSTEPS
_ROWBLK = 16
_RB = _ROWBLK * _NCLASS


def _fused(x_ref, adj_ref, w1_ref, b1_ref, w2_ref, b2_ref,
           fc1w_ref, fc1b_ref, fc2w_ref, fc2b_ref, fc3w_ref, fc3b_ref,
           out_ref, hflat_ref, acc_ref):
    f32 = jnp.float32
    g = pl.program_id(0)

    @pl.when(g == 0)
    def _gcn_and_flatten():
        adj = adj_ref[...]
        s1 = jnp.dot(x_ref[...], w1_ref[...], preferred_element_type=f32)
        h1 = jnp.maximum(
            jnp.dot(adj, s1, preferred_element_type=f32) + b1_ref[...], 0.0)
        s2 = jnp.dot(h1, w2_ref[...], preferred_element_type=f32)
        h2 = jnp.maximum(
            jnp.dot(adj, s2, preferred_element_type=f32) + b2_ref[...], 0.0)
        col = jax.lax.broadcasted_iota(jnp.int32, (_NCLASS, _RB), 1)
        tile = (col % _NCLASS == jax.lax.broadcasted_iota(
            jnp.int32, (_NCLASS, _RB), 0)).astype(f32)
        band = (jax.lax.broadcasted_iota(jnp.int32, (_ROWBLK, _RB), 1)
                // _NCLASS == jax.lax.broadcasted_iota(
                    jnp.int32, (_ROWBLK, _RB), 0))
        zero = jnp.zeros((_ROWBLK, _RB), f32)
        for r in range(_NNODES // _ROWBLK):
            expand = jnp.dot(h2[r * _ROWBLK:(r + 1) * _ROWBLK, :], tile,
                             preferred_element_type=f32)
            hflat_ref[:, r * _RB:(r + 1) * _RB] = jnp.sum(
                jnp.where(band, expand, zero), axis=0, keepdims=True)
        acc_ref[...] = jnp.zeros_like(acc_ref)

    acc_ref[...] += jax.lax.dot_general(
        hflat_ref[:, pl.ds(g * _CHUNK, _CHUNK)], fc1w_ref[...],
        (((1,), (1,)), ((), ())), preferred_element_type=f32)

    @pl.when(g == _NSTEPS - 1)
    def _head():
        f1 = jnp.maximum(acc_ref[...] + fc1b_ref[...], 0.0)
        f2 = jnp.sum(fc2w_ref[...] * f1, axis=1, keepdims=True)
        f2 = jnp.maximum(f2 + fc2b_ref[...], 0.0)
        f3 = jnp.sum(f2 * fc3w_ref[...], keepdims=True) + fc3b_ref[...]
        out_ref[...] = jax.nn.sigmoid(f3)


def kernel(x, adj, W1, b1, W2, b2, fc1_w, fc1_b, fc2_w, fc2_b, fc3_w, fc3_b):
    full = lambda a: pl.BlockSpec(a.shape, lambda g: (0,) * a.ndim)
    out = pl.pallas_call(
        _fused,
        grid=(_NSTEPS,),
        in_specs=[
            full(x), full(adj), full(W1), pl.BlockSpec((1, 256), lambda g: (0, 0)),
            full(W2), pl.BlockSpec((1, _NCLASS), lambda g: (0, 0)),
            pl.BlockSpec((128, _CHUNK), lambda g: (0, g)),
            pl.BlockSpec((1, 128), lambda g: (0, 0)),
            full(fc2_w), pl.BlockSpec((32, 1), lambda g: (0, 0)),
            pl.BlockSpec((32, 1), lambda g: (0, 0)),
            pl.BlockSpec((1, 1), lambda g: (0, 0)),
        ],
        out_specs=pl.BlockSpec((1, 1), lambda g: (0, 0)),
        out_shape=jax.ShapeDtypeStruct((1, 1), jnp.float32),
        scratch_shapes=[
            pltpu.VMEM((1, _FLAT), jnp.float32),
            pltpu.VMEM((1, 128), jnp.float32),
        ],
        compiler_params=pltpu.CompilerParams(
            dimension_semantics=("arbitrary",)),
    )(x, adj, W1, b1.reshape(1, -1), W2, b2.reshape(1, -1),
      fc1_w, fc1_b.reshape(1, -1), fc2_w, fc2_b.reshape(-1, 1),
      fc3_w.reshape(-1, 1), fc3_b.reshape(1, 1))
    return out.reshape(1)


# speedup vs baseline: 1.3950x; 1.1367x over previous
import jax
import jax.numpy as jnp
from jax.experimental import pallas as pl
from jax.experimental.pallas import tpu as pltpu

_NNODES = 208
_NFEAT = 512
_NHID = 256
_NCLASS = 64
_FLAT = _NNODES * _NCLASS
_NSLICES = 8
_ROWS = 128 // _NSLICES
_ROWBLK = 16
_RB = _ROWBLK * _NCLASS


def _fused(x_hbm, adj_hbm, w1_hbm, b1_ref, w2_ref, b2_ref,
           fc1w_hbm, fc1b_ref, fc2w_ref, fc2b_ref, fc3w_ref, fc3b_ref,
           out_ref, x_v, adj_v, w1_v, fc1_v, hflat_ref, sems, fsem):
    f32 = jnp.float32
    for k in range(_NSLICES):
        pltpu.make_async_copy(
            fc1w_hbm.at[k * _ROWS:(k + 1) * _ROWS, :],
            fc1_v.at[k * _ROWS:(k + 1) * _ROWS, :], sems.at[k]).start()
    cp_x = pltpu.make_async_copy(x_hbm, x_v, fsem.at[0])
    cp_w1 = pltpu.make_async_copy(w1_hbm, w1_v, fsem.at[1])
    cp_adj = pltpu.make_async_copy(adj_hbm, adj_v, fsem.at[2])
    cp_x.start()
    cp_w1.start()
    cp_adj.start()

    cp_x.wait()
    cp_w1.wait()
    s1 = jnp.dot(x_v[...], w1_v[...], preferred_element_type=f32)
    cp_adj.wait()
    adj = adj_v[...]
    h1 = jnp.maximum(jnp.dot(adj, s1, preferred_element_type=f32) + b1_ref[...], 0.0)
    s2 = jnp.dot(h1, w2_ref[...], preferred_element_type=f32)
    h2 = jnp.maximum(jnp.dot(adj, s2, preferred_element_type=f32) + b2_ref[...], 0.0)

    col = jax.lax.broadcasted_iota(jnp.int32, (_NCLASS, _RB), 1)
    tile = (col % _NCLASS == jax.lax.broadcasted_iota(
        jnp.int32, (_NCLASS, _RB), 0)).astype(f32)
    band = (jax.lax.broadcasted_iota(jnp.int32, (_ROWBLK, _RB), 1)
            // _NCLASS == jax.lax.broadcasted_iota(
                jnp.int32, (_ROWBLK, _RB), 0))
    zero = jnp.zeros((_ROWBLK, _RB), f32)
    for r in range(_NNODES // _ROWBLK):
        expand = jnp.dot(h2[r * _ROWBLK:(r + 1) * _ROWBLK, :], tile,
                         preferred_element_type=f32)
        hflat_ref[:, r * _RB:(r + 1) * _RB] = jnp.sum(
            jnp.where(band, expand, zero), axis=0, keepdims=True)
    hflat = hflat_ref[...]

    for k in range(_NSLICES):
        pltpu.make_async_copy(
            fc1w_hbm.at[k * _ROWS:(k + 1) * _ROWS, :],
            fc1_v.at[k * _ROWS:(k + 1) * _ROWS, :], sems.at[k]).wait()
    f1 = jax.lax.dot_general(hflat, fc1_v[...], (((1,), (1,)), ((), ())),
                             preferred_element_type=f32)
    f1 = jnp.maximum(f1 + fc1b_ref[...], 0.0)
    f2 = jnp.sum(fc2w_ref[...] * f1, axis=1, keepdims=True)
    f2 = jnp.maximum(f2 + fc2b_ref[...], 0.0)
    f3 = jnp.sum(f2 * fc3w_ref[...], keepdims=True) + fc3b_ref[...]
    out_ref[...] = jax.nn.sigmoid(f3)


def kernel(x, adj, W1, b1, W2, b2, fc1_w, fc1_b, fc2_w, fc2_b, fc3_w, fc3_b):
    hbm = pl.BlockSpec(memory_space=pltpu.MemorySpace.HBM)
    vmem = pl.BlockSpec(memory_space=pltpu.MemorySpace.VMEM)
    out = pl.pallas_call(
        _fused,
        in_specs=[hbm, hbm, hbm, vmem, vmem, vmem,
                  hbm, vmem, vmem, vmem, vmem, vmem],
        out_specs=vmem,
        out_shape=jax.ShapeDtypeStruct((1, 1), jnp.float32),
        scratch_shapes=[
            pltpu.VMEM((_NNODES, _NFEAT), jnp.float32),
            pltpu.VMEM((_NNODES, _NNODES), jnp.float32),
            pltpu.VMEM((_NFEAT, _NHID), jnp.float32),
            pltpu.VMEM((128, _FLAT), jnp.float32),
            pltpu.VMEM((1, _FLAT), jnp.float32),
            pltpu.SemaphoreType.DMA((_NSLICES,)),
            pltpu.SemaphoreType.DMA((3,)),
        ],
    )(x, adj, W1, b1.reshape(1, -1), W2, b2.reshape(1, -1),
      fc1_w, fc1_b.reshape(1, -1), fc2_w, fc2_b.reshape(-1, 1),
      fc3_w.reshape(-1, 1), fc3_b.reshape(1, 1))
    return out.reshape(1)
